# baseline (speedup 1.0000x reference)
import functools

import jax
import jax.numpy as jnp
from jax import lax
from jax.experimental import pallas as pl
from jax.experimental.pallas import tpu as pltpu
from jax.experimental.pallas import tpu_sc as plsc

N_GRAPHS = 64
BN_EPS = 1e-5


def _matmul_bias_body(x_ref, w_ref, b_ref, o_ref):
    o_ref[...] = (
        jnp.dot(x_ref[...], w_ref[...], preferred_element_type=jnp.float32)
        + b_ref[...]
    )


def _mlp_body(h_ref, p0_ref, p1_ref, w1_ref, b1_ref, g_ref, bb_ref, w2_ref,
              b2_ref, o_ref):
    z = h_ref[...] + p0_ref[...] + p1_ref[...]
    z = jnp.dot(z, w1_ref[...], preferred_element_type=jnp.float32) + b1_ref[...]
    mu = jnp.mean(z, axis=0, keepdims=True)
    zc = z - mu
    var = jnp.mean(zc * zc, axis=0, keepdims=True)
    z = g_ref[...] * zc * lax.rsqrt(var + BN_EPS) + bb_ref[...]
    z = jnp.maximum(z, 0.0)
    z = jnp.dot(z, w2_ref[...], preferred_element_type=jnp.float32) + b2_ref[...]
    o_ref[...] = jnp.maximum(z, 0.0)


def _pool_body(h_ref, b_ref, fcw_ref, fcb_ref, logits_ref, emb_ref):
    n = h_ref.shape[0]
    gids = lax.broadcasted_iota(jnp.int32, (N_GRAPHS, n), 0)
    mask = (gids == b_ref[...]).astype(jnp.float32)
    emb = jnp.dot(mask, h_ref[...], preferred_element_type=jnp.float32)
    emb_ref[...] = emb
    logits_ref[...] = (
        jnp.dot(emb, fcw_ref[...], preferred_element_type=jnp.float32)
        + fcb_ref[...]
    )


def _make_msg_kernel(n_pad, hid, n_chunks, k):
    info = plsc.get_sparse_core_info()
    nc, ns = info.num_cores, info.num_subcores
    nw = nc * ns
    q, r = divmod(n_chunks, nw)
    rows_per_tile = n_pad // ns
    zrows = rows_per_tile // 5
    assert rows_per_tile % 40 == 0 and n_pad % ns == 0

    mesh = plsc.VectorSubcoreMesh(core_axis_name="c", subcore_axis_name="s")

    @functools.partial(
        pl.kernel,
        out_type=jax.ShapeDtypeStruct((nc, n_pad, hid), jnp.float32),
        mesh=mesh,
        scratch_types=[
            pltpu.VMEM((2, k), jnp.int32),
            pltpu.VMEM((k, hid), jnp.float32),
            pltpu.VMEM((zrows, hid), jnp.float32),
            pltpu.VMEM_SHARED((n_pad, hid), jnp.float32),
            pltpu.SemaphoreType.DMA,
        ],
    )
    def msg(h_hbm, idx_hbm, e_hbm, zeros_hbm, out_hbm, idx_v, rows_v, zbuf,
            agg, sem):
        cid = lax.axis_index("c")
        sid = lax.axis_index("s")
        wid = sid * nc + cid

        pltpu.sync_copy(zeros_hbm, zbuf)
        for j in range(5):
            pltpu.sync_copy(zbuf, agg.at[pl.ds((sid * 5 + j) * zrows, zrows)])
        plsc.subcore_barrier()

        n_t = q + jnp.where(wid < r, 1, 0)

        def chunk_body(it, _):
            t = wid + it * nw
            pltpu.sync_copy(idx_hbm.at[t], idx_v)
            pltpu.sync_copy(e_hbm.at[pl.ds(t * k, k)], rows_v)
            pltpu.async_copy(h_hbm.at[idx_v.at[0]], rows_v, sem, add=True).wait()

            def relu_row(i, _):
                for j in range(hid // 16):
                    sl = pl.ds(j * 16, 16)
                    rows_v[i, sl] = jnp.maximum(rows_v[i, sl], 0.0)
                return 0

            lax.fori_loop(0, k, relu_row, 0)
            pltpu.sync_copy(rows_v, agg.at[idx_v.at[1]], add=True)
            return 0

        lax.fori_loop(0, n_t, chunk_body, 0)
        plsc.subcore_barrier()

        for j in range(5):
            r0 = (sid * 5 + j) * zrows
            pltpu.sync_copy(agg.at[pl.ds(r0, zrows)], zbuf)
            pltpu.sync_copy(zbuf, out_hbm.at[cid, pl.ds(r0, zrows)])

    return msg


def kernel(x, edge_index, batch, edge_attr, enc_W, enc_b, eenc_W, eenc_b,
           lin1_W, lin1_b, bn_g, bn_b, lin2_W, lin2_b, fc_W, fc_b):
    n, _ = x.shape
    e_cnt, _ = edge_attr.shape
    hid = enc_W.shape[1]
    n_layers = lin1_W.shape[0]
    n_out = fc_W.shape[1]
    f32 = jnp.float32

    K = 128
    NW = 32
    n_chunks = -(-e_cnt // (NW * K)) * NW
    e_pad_cnt = n_chunks * K
    n_pad = 10240

    h = pl.pallas_call(
        _matmul_bias_body,
        out_shape=jax.ShapeDtypeStruct((n, hid), f32),
    )(x, enc_W, enc_b.reshape(1, hid))

    ea_pad = jnp.pad(edge_attr, ((0, e_pad_cnt - e_cnt), (0, 0)))
    be = e_pad_cnt // 16
    e = pl.pallas_call(
        _matmul_bias_body,
        grid=(e_pad_cnt // be,),
        in_specs=[
            pl.BlockSpec((be, edge_attr.shape[1]), lambda i: (i, 0)),
            pl.BlockSpec(eenc_W.shape, lambda i: (0, 0)),
            pl.BlockSpec((1, hid), lambda i: (0, 0)),
        ],
        out_specs=pl.BlockSpec((be, hid), lambda i: (i, 0)),
        out_shape=jax.ShapeDtypeStruct((e_pad_cnt, hid), f32),
    )(ea_pad, eenc_W, eenc_b.reshape(1, hid))

    pad_n = e_pad_cnt - e_cnt
    src_p = jnp.concatenate([edge_index[0], jnp.zeros((pad_n,), jnp.int32)])
    dst_p = jnp.concatenate([edge_index[1], jnp.full((pad_n,), n, jnp.int32)])
    idx3 = (jnp.stack([src_p, dst_p])
            .reshape(2, n_chunks, K).transpose(1, 0, 2))
    zeros_init = jnp.zeros((n_pad // 16 // 5, hid), dtype=f32)

    msg_kernel = _make_msg_kernel(n_pad, hid, n_chunks, K)

    mlp = pl.pallas_call(
        _mlp_body,
        out_shape=jax.ShapeDtypeStruct((n, hid), f32),
    )

    for i in range(n_layers):
        parts = msg_kernel(h, idx3, e, zeros_init)
        parts = parts[:, :n]
        h = mlp(h, parts[0], parts[1],
                lin1_W[i], lin1_b[i].reshape(1, hid),
                bn_g[i].reshape(1, hid), bn_b[i].reshape(1, hid),
                lin2_W[i], lin2_b[i].reshape(1, hid))

    logits, emb = pl.pallas_call(
        _pool_body,
        out_shape=(
            jax.ShapeDtypeStruct((N_GRAPHS, n_out), f32),
            jax.ShapeDtypeStruct((N_GRAPHS, hid), f32),
        ),
    )(h, batch.reshape(1, n), fc_W, fc_b.reshape(1, n_out))

    return (logits, emb)


# speedup vs baseline: 1.2565x; 1.2565x over previous
import functools

import jax
import jax.numpy as jnp
from jax import lax
from jax.experimental import pallas as pl
from jax.experimental.pallas import tpu as pltpu
from jax.experimental.pallas import tpu_sc as plsc

N_GRAPHS = 64
BN_EPS = 1e-5


def _matmul_bias_body(x_ref, w_ref, b_ref, o_ref):
    o_ref[...] = (
        jnp.dot(x_ref[...], w_ref[...], preferred_element_type=jnp.float32)
        + b_ref[...]
    )


def _mlp_body(h_ref, p0_ref, p1_ref, w1_ref, b1_ref, g_ref, bb_ref, w2_ref,
              b2_ref, o_ref):
    z = h_ref[...] + p0_ref[...] + p1_ref[...]
    z = jnp.dot(z, w1_ref[...], preferred_element_type=jnp.float32) + b1_ref[...]
    mu = jnp.mean(z, axis=0, keepdims=True)
    zc = z - mu
    var = jnp.mean(zc * zc, axis=0, keepdims=True)
    z = g_ref[...] * zc * lax.rsqrt(var + BN_EPS) + bb_ref[...]
    z = jnp.maximum(z, 0.0)
    z = jnp.dot(z, w2_ref[...], preferred_element_type=jnp.float32) + b2_ref[...]
    o_ref[...] = jnp.maximum(z, 0.0)


def _pool_body(h_ref, b_ref, fcw_ref, fcb_ref, logits_ref, emb_ref):
    n = h_ref.shape[0]
    gids = lax.broadcasted_iota(jnp.int32, (N_GRAPHS, n), 0)
    mask = (gids == b_ref[...]).astype(jnp.float32)
    emb = jnp.dot(mask, h_ref[...], preferred_element_type=jnp.float32)
    emb_ref[...] = emb
    logits_ref[...] = (
        jnp.dot(emb, fcw_ref[...], preferred_element_type=jnp.float32)
        + fcb_ref[...]
    )


def _make_msg_kernel(n_pad, hid, n_chunks, k):
    info = plsc.get_sparse_core_info()
    nc, ns = info.num_cores, info.num_subcores
    nw = nc * ns
    q, r = divmod(n_chunks, nw)
    rows_per_tile = n_pad // ns
    zrows = rows_per_tile // 5
    assert rows_per_tile % 40 == 0 and n_pad % ns == 0

    mesh = plsc.VectorSubcoreMesh(core_axis_name="c", subcore_axis_name="s")

    @functools.partial(
        pl.kernel,
        out_type=jax.ShapeDtypeStruct((nc, n_pad, hid), jnp.float32),
        mesh=mesh,
        scratch_types=[
            pltpu.VMEM((2, k), jnp.int32),
            pltpu.VMEM((k, hid), jnp.float32),
            pltpu.VMEM((zrows, hid), jnp.float32),
            pltpu.VMEM_SHARED((n_pad, hid), jnp.float32),
            pltpu.SemaphoreType.DMA,
        ],
    )
    def msg(h_hbm, idx_hbm, e_hbm, zeros_hbm, out_hbm, idx_v, rows_v, zbuf,
            agg, sem):
        cid = lax.axis_index("c")
        sid = lax.axis_index("s")
        wid = sid * nc + cid

        pltpu.sync_copy(zeros_hbm, zbuf)
        for j in range(5):
            pltpu.sync_copy(zbuf, agg.at[pl.ds((sid * 5 + j) * zrows, zrows)])
        plsc.subcore_barrier()

        n_t = q + jnp.where(wid < r, 1, 0)

        def chunk_body(it, _):
            t = wid + it * nw
            pltpu.sync_copy(idx_hbm.at[t], idx_v)
            pltpu.sync_copy(e_hbm.at[pl.ds(t * k, k)], rows_v)
            pltpu.async_copy(h_hbm.at[idx_v.at[0]], rows_v, sem, add=True).wait()

            def relu_row(i, _):
                for j in range(hid // 16):
                    sl = pl.ds(j * 16, 16)
                    rows_v[i, sl] = jnp.maximum(rows_v[i, sl], 0.0)
                return 0

            lax.fori_loop(0, k, relu_row, 0)
            pltpu.sync_copy(rows_v, agg.at[idx_v.at[1]], add=True)
            return 0

        lax.fori_loop(0, n_t, chunk_body, 0)
        plsc.subcore_barrier()

        for j in range(5):
            r0 = (sid * 5 + j) * zrows
            pltpu.sync_copy(agg.at[pl.ds(r0, zrows)], zbuf)
            pltpu.sync_copy(zbuf, out_hbm.at[cid, pl.ds(r0, zrows)])

    return msg


def kernel(x, edge_index, batch, edge_attr, enc_W, enc_b, eenc_W, eenc_b,
           lin1_W, lin1_b, bn_g, bn_b, lin2_W, lin2_b, fc_W, fc_b):
    n, _ = x.shape
    e_cnt, _ = edge_attr.shape
    hid = enc_W.shape[1]
    n_layers = lin1_W.shape[0]
    n_out = fc_W.shape[1]
    f32 = jnp.float32

    K = 128
    NW = 32
    n_chunks = -(-e_cnt // (NW * K)) * NW
    e_pad_cnt = n_chunks * K
    n_pad = 10240

    h = pl.pallas_call(
        _matmul_bias_body,
        out_shape=jax.ShapeDtypeStruct((n, hid), f32),
    )(x, enc_W, enc_b.reshape(1, hid))

    ea_pad = jnp.pad(edge_attr, ((0, e_pad_cnt - e_cnt), (0, 0)))
    be = e_pad_cnt // 16
    e = pl.pallas_call(
        _matmul_bias_body,
        grid=(e_pad_cnt // be,),
        in_specs=[
            pl.BlockSpec((be, edge_attr.shape[1]), lambda i: (i, 0)),
            pl.BlockSpec(eenc_W.shape, lambda i: (0, 0)),
            pl.BlockSpec((1, hid), lambda i: (0, 0)),
        ],
        out_specs=pl.BlockSpec((be, hid), lambda i: (i, 0)),
        out_shape=jax.ShapeDtypeStruct((e_pad_cnt, hid), f32),
    )(ea_pad, eenc_W, eenc_b.reshape(1, hid))

    pad_n = e_pad_cnt - e_cnt
    pad_ar = jnp.arange(pad_n, dtype=jnp.int32)
    src_p = jnp.concatenate([edge_index[0], pad_ar % n])
    dst_p = jnp.concatenate([edge_index[1], n + pad_ar % (n_pad - n)])
    idx3 = (jnp.stack([src_p, dst_p])
            .reshape(2, n_chunks, K).transpose(1, 0, 2))
    zeros_init = jnp.zeros((n_pad // 16 // 5, hid), dtype=f32)

    msg_kernel = _make_msg_kernel(n_pad, hid, n_chunks, K)

    mlp = pl.pallas_call(
        _mlp_body,
        out_shape=jax.ShapeDtypeStruct((n, hid), f32),
    )

    for i in range(n_layers):
        parts = msg_kernel(h, idx3, e, zeros_init)
        parts = parts[:, :n]
        h = mlp(h, parts[0], parts[1],
                lin1_W[i], lin1_b[i].reshape(1, hid),
                bn_g[i].reshape(1, hid), bn_b[i].reshape(1, hid),
                lin2_W[i], lin2_b[i].reshape(1, hid))

    logits, emb = pl.pallas_call(
        _pool_body,
        out_shape=(
            jax.ShapeDtypeStruct((N_GRAPHS, n_out), f32),
            jax.ShapeDtypeStruct((N_GRAPHS, hid), f32),
        ),
    )(h, batch.reshape(1, n), fc_W, fc_b.reshape(1, n_out))

    return (logits, emb)


# speedup vs baseline: 1.6603x; 1.3213x over previous
import functools

import jax
import jax.numpy as jnp
from jax import lax
from jax.experimental import pallas as pl
from jax.experimental.pallas import tpu as pltpu
from jax.experimental.pallas import tpu_sc as plsc

N_GRAPHS = 64
BN_EPS = 1e-5


def _matmul_bias_body(x_ref, w_ref, b_ref, o_ref):
    o_ref[...] = (
        jnp.dot(x_ref[...], w_ref[...], preferred_element_type=jnp.float32)
        + b_ref[...]
    )


def _mlp_body(h_ref, p0_ref, p1_ref, w1_ref, b1_ref, g_ref, bb_ref, w2_ref,
              b2_ref, o_ref):
    z = h_ref[...] + p0_ref[...] + p1_ref[...]
    z = jnp.dot(z, w1_ref[...], preferred_element_type=jnp.float32) + b1_ref[...]
    mu = jnp.mean(z, axis=0, keepdims=True)
    zc = z - mu
    var = jnp.mean(zc * zc, axis=0, keepdims=True)
    z = g_ref[...] * zc * lax.rsqrt(var + BN_EPS) + bb_ref[...]
    z = jnp.maximum(z, 0.0)
    z = jnp.dot(z, w2_ref[...], preferred_element_type=jnp.float32) + b2_ref[...]
    o_ref[...] = jnp.maximum(z, 0.0)


def _pool_body(h_ref, b_ref, fcw_ref, fcb_ref, logits_ref, emb_ref):
    n = h_ref.shape[0]
    gids = lax.broadcasted_iota(jnp.int32, (N_GRAPHS, n), 0)
    mask = (gids == b_ref[...]).astype(jnp.float32)
    emb = jnp.dot(mask, h_ref[...], preferred_element_type=jnp.float32)
    emb_ref[...] = emb
    logits_ref[...] = (
        jnp.dot(emb, fcw_ref[...], preferred_element_type=jnp.float32)
        + fcb_ref[...]
    )


def _make_msg_kernel(n_pad, hid, n_chunks, k):
    info = plsc.get_sparse_core_info()
    nc, ns = info.num_cores, info.num_subcores
    nw = nc * ns
    q, r = divmod(n_chunks, nw)
    rows_per_tile = n_pad // ns
    zrows = rows_per_tile // 5
    assert rows_per_tile % 40 == 0 and n_pad % ns == 0

    mesh = plsc.VectorSubcoreMesh(core_axis_name="c", subcore_axis_name="s")

    @functools.partial(
        pl.kernel,
        out_type=jax.ShapeDtypeStruct((nc, n_pad, hid), jnp.float32),
        mesh=mesh,
        scratch_types=[
            pltpu.VMEM((2, k), jnp.int32),
            pltpu.VMEM((k, hid), jnp.float32),
            pltpu.VMEM((2, k), jnp.int32),
            pltpu.VMEM((k, hid), jnp.float32),
            pltpu.VMEM_SHARED((n_pad, hid), jnp.float32),
            pltpu.SemaphoreType.DMA,
            pltpu.SemaphoreType.DMA,
            pltpu.SemaphoreType.DMA,
            pltpu.SemaphoreType.DMA,
        ],
    )
    def msg(h_hbm, idx_hbm, e_hbm, zeros_hbm, out_hbm, idx_v, rows_v,
            idx_v1, rows_v1, agg, sem, sem1, sem_s0, sem_s1):
        zbuf = rows_v1
        cid = lax.axis_index("c")
        sid = lax.axis_index("s")
        wid = sid * nc + cid

        pltpu.sync_copy(zeros_hbm, zbuf)
        for j in range(5):
            pltpu.sync_copy(zbuf, agg.at[pl.ds((sid * 5 + j) * zrows, zrows)])
        plsc.subcore_barrier()

        n_t = q + jnp.where(wid < r, 1, 0)

        def relu_rows(rv):
            def relu_row(i, _):
                for j in range(hid // 16):
                    sl = pl.ds(j * 16, 16)
                    rv[i, sl] = jnp.maximum(rv[i, sl], 0.0)
                return 0
            lax.fori_loop(0, k, relu_row, 0)

        bufs = ((idx_v, rows_v, sem, sem_s0), (idx_v1, rows_v1, sem1, sem_s1))

        def group_body(gi, _):
            g_d = []
            for b in range(2):
                iv, rv, sg, ss = bufs[b]
                t = wid + (2 * gi + b) * nw
                @pl.when(gi > 0)
                def _drain():
                    pltpu.make_async_copy(e_hbm.at[pl.ds(0, k)], rv, ss).wait()
                pltpu.sync_copy(idx_hbm.at[t], iv)
                pltpu.sync_copy(e_hbm.at[pl.ds(t * k, k)], rv)
                g_d.append(pltpu.async_copy(h_hbm.at[iv.at[0]], rv, sg,
                                            add=True))
            for b in range(2):
                iv, rv, sg, ss = bufs[b]
                g_d[b].wait()
                relu_rows(rv)
                pltpu.async_copy(rv, agg.at[iv.at[1]], ss, add=True)
            return 0

        lax.fori_loop(0, n_t // 2, group_body, 0)
        for b in range(2):
            iv, rv, sg, ss = bufs[b]
            pltpu.make_async_copy(e_hbm.at[pl.ds(0, k)], rv, ss).wait()
        plsc.subcore_barrier()

        for j in range(5):
            r0 = (sid * 5 + j) * zrows
            pltpu.sync_copy(agg.at[pl.ds(r0, zrows)], zbuf)
            pltpu.sync_copy(zbuf, out_hbm.at[cid, pl.ds(r0, zrows)])

    return msg


def kernel(x, edge_index, batch, edge_attr, enc_W, enc_b, eenc_W, eenc_b,
           lin1_W, lin1_b, bn_g, bn_b, lin2_W, lin2_b, fc_W, fc_b):
    n, _ = x.shape
    e_cnt, _ = edge_attr.shape
    hid = enc_W.shape[1]
    n_layers = lin1_W.shape[0]
    n_out = fc_W.shape[1]
    f32 = jnp.float32

    K = 128
    NW = 32
    n_chunks = -(-e_cnt // (NW * K)) * NW
    e_pad_cnt = n_chunks * K
    n_pad = 10240

    h = pl.pallas_call(
        _matmul_bias_body,
        out_shape=jax.ShapeDtypeStruct((n, hid), f32),
    )(x, enc_W, enc_b.reshape(1, hid))

    ea_pad = jnp.pad(edge_attr, ((0, e_pad_cnt - e_cnt), (0, 0)))
    be = e_pad_cnt // 16
    e = pl.pallas_call(
        _matmul_bias_body,
        grid=(e_pad_cnt // be,),
        in_specs=[
            pl.BlockSpec((be, edge_attr.shape[1]), lambda i: (i, 0)),
            pl.BlockSpec(eenc_W.shape, lambda i: (0, 0)),
            pl.BlockSpec((1, hid), lambda i: (0, 0)),
        ],
        out_specs=pl.BlockSpec((be, hid), lambda i: (i, 0)),
        out_shape=jax.ShapeDtypeStruct((e_pad_cnt, hid), f32),
    )(ea_pad, eenc_W, eenc_b.reshape(1, hid))

    pad_n = e_pad_cnt - e_cnt
    pad_ar = jnp.arange(pad_n, dtype=jnp.int32)
    src_p = jnp.concatenate([edge_index[0], pad_ar % n])
    dst_p = jnp.concatenate([edge_index[1], n + pad_ar % (n_pad - n)])
    idx3 = (jnp.stack([src_p, dst_p])
            .reshape(2, n_chunks, K).transpose(1, 0, 2))
    zeros_init = jnp.zeros((n_pad // 16 // 5, hid), dtype=f32)

    msg_kernel = _make_msg_kernel(n_pad, hid, n_chunks, K)

    mlp = pl.pallas_call(
        _mlp_body,
        out_shape=jax.ShapeDtypeStruct((n, hid), f32),
    )

    for i in range(n_layers):
        parts = msg_kernel(h, idx3, e, zeros_init)
        parts = parts[:, :n]
        h = mlp(h, parts[0], parts[1],
                lin1_W[i], lin1_b[i].reshape(1, hid),
                bn_g[i].reshape(1, hid), bn_b[i].reshape(1, hid),
                lin2_W[i], lin2_b[i].reshape(1, hid))

    logits, emb = pl.pallas_call(
        _pool_body,
        out_shape=(
            jax.ShapeDtypeStruct((N_GRAPHS, n_out), f32),
            jax.ShapeDtypeStruct((N_GRAPHS, hid), f32),
        ),
    )(h, batch.reshape(1, n), fc_W, fc_b.reshape(1, n_out))

    return (logits, emb)
